# baseline (speedup 1.0000x reference)
import jax
import jax.numpy as jnp
from jax import lax
from jax.experimental import pallas as pl
from jax.experimental.pallas import tpu as pltpu
from jax.experimental.pallas import tpu_sc as plsc

BATCH = 16384
EMBED_DIM = 32
_GW = 4 * EMBED_DIM

_UE_LOG = 17
_IE_LOG = 14
_UE = 1 << _UE_LOG
_IE = 1 << _IE_LOG

_NC = 2
_NS = 16
_NW = _NC * _NS
_B_PER_W = BATCH // _NW
_CHUNK = 128
_NCHUNK = _B_PER_W // _CHUNK

_PRE_BLK = 8192


def _prepass_body(x0, x1, x2, x3, x4, x5, x6, x7, eye_ref, o_ref):
    e = eye_ref[...]
    dn = (((0,), (0,)), ((), ()))
    xs = (x0, x1, x2, x3, x4, x5, x6, x7)
    u32 = jnp.uint32
    w = []
    for x in xs:
        t = lax.dot_general(x[...], e, dn, preferred_element_type=jnp.float32)
        b = t.astype(jnp.bfloat16)
        w.append(lax.convert_element_type(
            lax.bitcast_convert_type(b, jnp.uint16), u32))
    pairs = [
        lax.bitcast_convert_type(
            jnp.bitwise_or(jnp.left_shift(w[2 * s + 1], jnp.uint32(16)),
                           w[2 * s]),
            jnp.float32)
        for s in range(4)
    ]
    o_ref[...] = jnp.concatenate(pairs, axis=1)


def _prepass(tabT, ebits, dep=None):
    eq = 1 << ebits
    nblk = eq // _PRE_BLK
    n = tabT.shape[1]
    max_blk = (n - 1) // _PRE_BLK

    def make_im(s):
        return lambda b: (0, jnp.minimum(s * nblk + b, max_blk))

    eye = jnp.eye(EMBED_DIM, dtype=jnp.float32)
    if dep is not None:
        eye = eye + dep
    return pl.pallas_call(
        _prepass_body,
        grid=(nblk,),
        in_specs=[
            pl.BlockSpec((EMBED_DIM, _PRE_BLK), make_im(s)) for s in range(8)
        ] + [pl.BlockSpec((EMBED_DIM, EMBED_DIM), lambda b: (0, 0))],
        out_specs=pl.BlockSpec((_PRE_BLK, _GW), lambda b: (b, 0)),
        out_shape=jax.ShapeDtypeStruct((eq, _GW), jnp.float32),
    )(*([tabT] * 8), eye)


def _sc_gather_body(qmask, ids_hbm, tab_hbm, out_hbm, q_v, buf, buf2,
                    sem, sem2):
    wid = lax.axis_index("s") * _NC + lax.axis_index("c")
    base = wid * _B_PER_W
    pltpu.sync_copy(ids_hbm.at[pl.ds(base, _B_PER_W)], q_v)

    def toq(g, carry):
        sl = pl.ds(g * 16, 16)
        q_v[sl] = lax.bitwise_and(q_v[sl], qmask)
        return carry
    lax.fori_loop(0, _B_PER_W // 16, toq, 0)

    bufs = (buf, buf2)
    sems = (sem, sem2)
    cps = [None, None]
    cps[0] = pltpu.async_copy(
        tab_hbm.at[q_v.at[pl.ds(0, _CHUNK)]], bufs[0], sems[0])
    for c in range(_NCHUNK):
        if c + 1 < _NCHUNK:
            nsl = pl.ds((c + 1) * _CHUNK, _CHUNK)
            cps[(c + 1) % 2] = pltpu.async_copy(
                tab_hbm.at[q_v.at[nsl]], bufs[(c + 1) % 2], sems[(c + 1) % 2])
        cps[c % 2].wait()
        pltpu.sync_copy(
            bufs[c % 2], out_hbm.at[pl.ds(base + c * _CHUNK, _CHUNK)])


def _sc_gather(ids, tab4, qmask):
    mesh = plsc.VectorSubcoreMesh(
        core_axis_name="c", subcore_axis_name="s",
        num_cores=_NC, num_subcores=_NS)
    f = pl.kernel(
        lambda *a: _sc_gather_body(qmask, *a),
        out_type=jax.ShapeDtypeStruct((BATCH, _GW), jnp.float32),
        mesh=mesh,
        scratch_types=(
            pltpu.VMEM((_B_PER_W,), jnp.int32),
            pltpu.VMEM((_CHUNK, _GW), jnp.float32),
            pltpu.VMEM((_CHUNK, _GW), jnp.float32),
            pltpu.SemaphoreType.DMA,
            pltpu.SemaphoreType.DMA,
        ),
        compiler_params=pltpu.CompilerParams(use_tc_tiling_on_sc=True),
    )
    return f(ids, tab4)


_MLP_BLK = 2048


def _unpack(packed, ids, ebits):
    u32 = jnp.uint32
    w = lax.bitcast_convert_type(packed, u32)
    half = lax.bitwise_and(lax.shift_right_logical(ids, ebits), 1)
    hi = jnp.bitwise_and(w, jnp.uint32(0xFFFF0000))
    lo = jnp.left_shift(w, jnp.uint32(16))
    return lax.bitcast_convert_type(jnp.where(half == 1, hi, lo), jnp.float32)


def _mlp_body(u4_ref, i4_ref, uid_ref, iid_ref,
              w1a_ref, w1b_ref, b1_ref, w2_ref, b2_ref,
              w3_ref, b3_ref, w4_ref, b4_ref, out_ref):
    f32 = jnp.float32
    uid = uid_ref[...]
    iid = iid_ref[...]
    su = lax.shift_right_logical(uid, _UE_LOG + 1)
    si = lax.shift_right_logical(iid, _IE_LOG + 1)
    u4 = _unpack(u4_ref[...].reshape(_MLP_BLK, _GW), uid, _UE_LOG)
    i4 = _unpack(i4_ref[...].reshape(_MLP_BLK, _GW), iid, _IE_LOG)
    lane_grp = lax.shift_right_logical(
        lax.broadcasted_iota(jnp.int32, (_MLP_BLK, _GW), 1), 5)
    um = jnp.where(lane_grp == su, u4, 0.0)
    im = jnp.where(lane_grp == si, i4, 0.0)
    h = (jnp.dot(um, w1a_ref[...], preferred_element_type=f32)
         + jnp.dot(im, w1b_ref[...], preferred_element_type=f32)
         + b1_ref[...])
    h = jnp.maximum(h, 0.0)
    h = jnp.dot(h, w2_ref[...], preferred_element_type=f32) + b2_ref[...]
    h = jnp.maximum(h, 0.0)
    h = jnp.dot(h, w3_ref[...], preferred_element_type=f32) + b3_ref[...]
    h = jnp.maximum(h, 0.0)
    z = jnp.dot(h, w4_ref[...], preferred_element_type=f32) + b4_ref[...]
    out_ref[...] = jax.nn.sigmoid(z)


def _mlp(u4, i4, user_ids, item_ids, W1, b1, W2, b2, W3, b3, W4, b4):
    w1a = jnp.concatenate([W1[:EMBED_DIM]] * 4, axis=0)
    w1b = jnp.concatenate([W1[EMBED_DIM:]] * 4, axis=0)
    grid = BATCH // _MLP_BLK
    full = lambda a: pl.BlockSpec(a.shape, lambda b: (0,) * a.ndim)
    out = pl.pallas_call(
        _mlp_body,
        grid=(grid,),
        in_specs=[
            pl.BlockSpec((_MLP_BLK // 8, 8, _GW), lambda b: (b, 0, 0)),
            pl.BlockSpec((_MLP_BLK // 8, 8, _GW), lambda b: (b, 0, 0)),
            pl.BlockSpec((_MLP_BLK, 1), lambda b: (b, 0)),
            pl.BlockSpec((_MLP_BLK, 1), lambda b: (b, 0)),
            full(w1a), full(w1b),
            pl.BlockSpec((1, 64), lambda b: (0, 0)),
            full(W2),
            pl.BlockSpec((1, 32), lambda b: (0, 0)),
            full(W3),
            pl.BlockSpec((1, 16), lambda b: (0, 0)),
            full(W4),
            pl.BlockSpec((1, 1), lambda b: (0, 0)),
        ],
        out_specs=pl.BlockSpec((_MLP_BLK, 1), lambda b: (b, 0)),
        out_shape=jax.ShapeDtypeStruct((BATCH, 1), jnp.float32),
    )(u4.reshape(BATCH // 8, 8, _GW), i4.reshape(BATCH // 8, 8, _GW),
      user_ids.reshape(BATCH, 1), item_ids.reshape(BATCH, 1),
      w1a, w1b, b1.reshape(1, 64), W2, b2.reshape(1, 32),
      W3, b3.reshape(1, 16), W4, b4.reshape(1, 1))
    return out[:, 0]


def kernel(user_ids, item_ids, user_table, item_table,
           W1, b1, W2, b2, W3, b3, W4, b4):
    itab4 = _prepass(item_table.T, _IE_LOG)
    i4 = _sc_gather(item_ids, itab4, _IE - 1)
    dep = itab4[:EMBED_DIM, :EMBED_DIM] * 0.0
    utab4 = _prepass(user_table.T, _UE_LOG, dep)
    u4 = _sc_gather(user_ids, utab4, _UE - 1)
    return _mlp(u4, i4, user_ids, item_ids, W1, b1, W2, b2, W3, b3, W4, b4)


# speedup vs baseline: 1.0269x; 1.0269x over previous
import jax
import jax.numpy as jnp
from jax import lax
from jax.experimental import pallas as pl
from jax.experimental.pallas import tpu as pltpu
from jax.experimental.pallas import tpu_sc as plsc

BATCH = 16384
EMBED_DIM = 32
_GW = 4 * EMBED_DIM

_UE_LOG = 17
_IE_LOG = 14
_UE = 1 << _UE_LOG
_IE = 1 << _IE_LOG

_NC = 2
_NS = 16
_NW = _NC * _NS
_B_PER_W = BATCH // _NW
_CHUNK = 128
_NCHUNK = _B_PER_W // _CHUNK

_PRE_BLK = 8192


def _prepass_body(x0, x1, x2, x3, x4, x5, x6, x7, eye_ref, o_ref):
    e = eye_ref[...]
    dn = (((0,), (0,)), ((), ()))
    xs = (x0, x1, x2, x3, x4, x5, x6, x7)
    u32 = jnp.uint32
    w = []
    for x in xs:
        t = lax.dot_general(x[...], e, dn, preferred_element_type=jnp.float32)
        b = t.astype(jnp.bfloat16)
        w.append(lax.convert_element_type(
            lax.bitcast_convert_type(b, jnp.uint16), u32))
    pairs = [
        lax.bitcast_convert_type(
            jnp.bitwise_or(jnp.left_shift(w[2 * s + 1], jnp.uint32(16)),
                           w[2 * s]),
            jnp.float32)
        for s in range(4)
    ]
    o_ref[...] = jnp.concatenate(pairs, axis=1)


def _prepass(tabT, ebits, blk=_PRE_BLK):
    eq = 1 << ebits
    nblk = eq // blk
    n = tabT.shape[1]
    max_blk = (n - 1) // blk

    def make_im(s):
        return lambda b: (0, jnp.minimum(s * nblk + b, max_blk))

    return pl.pallas_call(
        _prepass_body,
        grid=(nblk,),
        in_specs=[
            pl.BlockSpec((EMBED_DIM, blk), make_im(s)) for s in range(8)
        ] + [pl.BlockSpec((EMBED_DIM, EMBED_DIM), lambda b: (0, 0))],
        out_specs=pl.BlockSpec((blk, _GW), lambda b: (b, 0)),
        out_shape=jax.ShapeDtypeStruct((eq, _GW), jnp.float32),
    )(*([tabT] * 8), jnp.eye(EMBED_DIM, dtype=jnp.float32))


def _sc_gather_body(qmask, ids_hbm, tab_hbm, out_hbm, q_v, buf, buf2,
                    sem, sem2):
    wid = lax.axis_index("s") * _NC + lax.axis_index("c")
    base = wid * _B_PER_W
    pltpu.sync_copy(ids_hbm.at[pl.ds(base, _B_PER_W)], q_v)

    def toq(g, carry):
        sl = pl.ds(g * 16, 16)
        q_v[sl] = lax.bitwise_and(q_v[sl], qmask)
        return carry
    lax.fori_loop(0, _B_PER_W // 16, toq, 0)

    bufs = (buf, buf2)
    sems = (sem, sem2)
    cps = [None, None]
    cps[0] = pltpu.async_copy(
        tab_hbm.at[q_v.at[pl.ds(0, _CHUNK)]], bufs[0], sems[0])
    for c in range(_NCHUNK):
        if c + 1 < _NCHUNK:
            nsl = pl.ds((c + 1) * _CHUNK, _CHUNK)
            cps[(c + 1) % 2] = pltpu.async_copy(
                tab_hbm.at[q_v.at[nsl]], bufs[(c + 1) % 2], sems[(c + 1) % 2])
        cps[c % 2].wait()
        pltpu.sync_copy(
            bufs[c % 2], out_hbm.at[pl.ds(base + c * _CHUNK, _CHUNK)])


def _sc_gather(ids, tab4, qmask):
    mesh = plsc.VectorSubcoreMesh(
        core_axis_name="c", subcore_axis_name="s",
        num_cores=_NC, num_subcores=_NS)
    f = pl.kernel(
        lambda *a: _sc_gather_body(qmask, *a),
        out_type=jax.ShapeDtypeStruct((BATCH, _GW), jnp.float32),
        mesh=mesh,
        scratch_types=(
            pltpu.VMEM((_B_PER_W,), jnp.int32),
            pltpu.VMEM((_CHUNK, _GW), jnp.float32),
            pltpu.VMEM((_CHUNK, _GW), jnp.float32),
            pltpu.SemaphoreType.DMA,
            pltpu.SemaphoreType.DMA,
        ),
        compiler_params=pltpu.CompilerParams(use_tc_tiling_on_sc=True),
    )
    return f(ids, tab4)


_MLP_BLK = 2048


def _unpack(packed, ids, ebits):
    u32 = jnp.uint32
    w = lax.bitcast_convert_type(packed, u32)
    half = lax.bitwise_and(lax.shift_right_logical(ids, ebits), 1)
    hi = jnp.bitwise_and(w, jnp.uint32(0xFFFF0000))
    lo = jnp.left_shift(w, jnp.uint32(16))
    return lax.bitcast_convert_type(jnp.where(half == 1, hi, lo), jnp.float32)


def _mlp_body(u4_ref, i4_ref, uid_ref, iid_ref,
              w1a_ref, w1b_ref, b1_ref, w2_ref, b2_ref,
              w3_ref, b3_ref, w4_ref, b4_ref, out_ref):
    f32 = jnp.float32
    uid = uid_ref[...]
    iid = iid_ref[...]
    su = lax.shift_right_logical(uid, _UE_LOG + 1)
    si = lax.shift_right_logical(iid, _IE_LOG + 1)
    u4 = _unpack(u4_ref[...].reshape(_MLP_BLK, _GW), uid, _UE_LOG)
    i4 = _unpack(i4_ref[...].reshape(_MLP_BLK, _GW), iid, _IE_LOG)
    lane_grp = lax.shift_right_logical(
        lax.broadcasted_iota(jnp.int32, (_MLP_BLK, _GW), 1), 5)
    um = jnp.where(lane_grp == su, u4, 0.0)
    im = jnp.where(lane_grp == si, i4, 0.0)
    h = (jnp.dot(um, w1a_ref[...], preferred_element_type=f32)
         + jnp.dot(im, w1b_ref[...], preferred_element_type=f32)
         + b1_ref[...])
    h = jnp.maximum(h, 0.0)
    h = jnp.dot(h, w2_ref[...], preferred_element_type=f32) + b2_ref[...]
    h = jnp.maximum(h, 0.0)
    h = jnp.dot(h, w3_ref[...], preferred_element_type=f32) + b3_ref[...]
    h = jnp.maximum(h, 0.0)
    z = jnp.dot(h, w4_ref[...], preferred_element_type=f32) + b4_ref[...]
    out_ref[...] = jax.nn.sigmoid(z)


def _mlp(u4, i4, user_ids, item_ids, W1, b1, W2, b2, W3, b3, W4, b4):
    w1a = jnp.concatenate([W1[:EMBED_DIM]] * 4, axis=0)
    w1b = jnp.concatenate([W1[EMBED_DIM:]] * 4, axis=0)
    grid = BATCH // _MLP_BLK
    full = lambda a: pl.BlockSpec(a.shape, lambda b: (0,) * a.ndim)
    out = pl.pallas_call(
        _mlp_body,
        grid=(grid,),
        in_specs=[
            pl.BlockSpec((_MLP_BLK // 8, 8, _GW), lambda b: (b, 0, 0)),
            pl.BlockSpec((_MLP_BLK // 8, 8, _GW), lambda b: (b, 0, 0)),
            pl.BlockSpec((_MLP_BLK, 1), lambda b: (b, 0)),
            pl.BlockSpec((_MLP_BLK, 1), lambda b: (b, 0)),
            full(w1a), full(w1b),
            pl.BlockSpec((1, 64), lambda b: (0, 0)),
            full(W2),
            pl.BlockSpec((1, 32), lambda b: (0, 0)),
            full(W3),
            pl.BlockSpec((1, 16), lambda b: (0, 0)),
            full(W4),
            pl.BlockSpec((1, 1), lambda b: (0, 0)),
        ],
        out_specs=pl.BlockSpec((_MLP_BLK, 1), lambda b: (b, 0)),
        out_shape=jax.ShapeDtypeStruct((BATCH, 1), jnp.float32),
    )(u4.reshape(BATCH // 8, 8, _GW), i4.reshape(BATCH // 8, 8, _GW),
      user_ids.reshape(BATCH, 1), item_ids.reshape(BATCH, 1),
      w1a, w1b, b1.reshape(1, 64), W2, b2.reshape(1, 32),
      W3, b3.reshape(1, 16), W4, b4.reshape(1, 1))
    return out[:, 0]


def kernel(user_ids, item_ids, user_table, item_table,
           W1, b1, W2, b2, W3, b3, W4, b4):
    itab4 = _prepass(item_table.T, _IE_LOG, blk=2048)
    i4 = _sc_gather(item_ids, itab4, _IE - 1)
    utab4 = _prepass(user_table.T, _UE_LOG)
    u4 = _sc_gather(user_ids, utab4, _UE - 1)
    return _mlp(u4, i4, user_ids, item_ids, W1, b1, W2, b2, W3, b3, W4, b4)


# speedup vs baseline: 1.0298x; 1.0028x over previous
import jax
import jax.numpy as jnp
from jax import lax
from jax.experimental import pallas as pl
from jax.experimental.pallas import tpu as pltpu
from jax.experimental.pallas import tpu_sc as plsc

BATCH = 16384
EMBED_DIM = 32
_GW = 4 * EMBED_DIM

_UE_LOG = 17
_IE_LOG = 14
_UE = 1 << _UE_LOG
_IE = 1 << _IE_LOG

_NC = 2
_NS = 16
_NW = _NC * _NS
_B_PER_W = BATCH // _NW
_CHUNK = 128
_NCHUNK = _B_PER_W // _CHUNK

_PRE_BLK = 8192


def _prepass_body(x0, x1, x2, x3, x4, x5, x6, x7, eye_ref, o_ref):
    e = eye_ref[...]
    dn = (((0,), (0,)), ((), ()))
    xs = (x0, x1, x2, x3, x4, x5, x6, x7)
    u32 = jnp.uint32
    w = []
    for x in xs:
        t = lax.dot_general(x[...], e, dn, preferred_element_type=jnp.float32)
        b = t.astype(jnp.bfloat16)
        w.append(lax.convert_element_type(
            lax.bitcast_convert_type(b, jnp.uint16), u32))
    pairs = [
        lax.bitcast_convert_type(
            jnp.bitwise_or(jnp.left_shift(w[2 * s + 1], jnp.uint32(16)),
                           w[2 * s]),
            jnp.float32)
        for s in range(4)
    ]
    o_ref[...] = jnp.concatenate(pairs, axis=1)


def _prepass(tabT, ebits, blk=_PRE_BLK):
    eq = 1 << ebits
    nblk = eq // blk
    n = tabT.shape[1]
    max_blk = (n - 1) // blk

    def make_im(s):
        return lambda b: (0, jnp.minimum(s * nblk + b, max_blk))

    return pl.pallas_call(
        _prepass_body,
        grid=(nblk,),
        in_specs=[
            pl.BlockSpec((EMBED_DIM, blk), make_im(s)) for s in range(8)
        ] + [pl.BlockSpec((EMBED_DIM, EMBED_DIM), lambda b: (0, 0))],
        out_specs=pl.BlockSpec((blk, _GW), lambda b: (b, 0)),
        out_shape=jax.ShapeDtypeStruct((eq, _GW), jnp.float32),
    )(*([tabT] * 8), jnp.eye(EMBED_DIM, dtype=jnp.float32))


def _sc_gather_body(qmask, ids_hbm, tab_hbm, out_hbm, q_v, buf, buf2,
                    sem, sem2):
    wid = lax.axis_index("s") * _NC + lax.axis_index("c")
    base = wid * _B_PER_W
    pltpu.sync_copy(ids_hbm.at[pl.ds(base, _B_PER_W)], q_v)

    def toq(g, carry):
        sl = pl.ds(g * 16, 16)
        q_v[sl] = lax.bitwise_and(q_v[sl], qmask)
        return carry
    lax.fori_loop(0, _B_PER_W // 16, toq, 0)

    bufs = (buf, buf2)
    sems = (sem, sem2)
    cps = [None, None]
    cps[0] = pltpu.async_copy(
        tab_hbm.at[q_v.at[pl.ds(0, _CHUNK)]], bufs[0], sems[0])
    for c in range(_NCHUNK):
        if c + 1 < _NCHUNK:
            nsl = pl.ds((c + 1) * _CHUNK, _CHUNK)
            cps[(c + 1) % 2] = pltpu.async_copy(
                tab_hbm.at[q_v.at[nsl]], bufs[(c + 1) % 2], sems[(c + 1) % 2])
        cps[c % 2].wait()
        pltpu.sync_copy(
            bufs[c % 2], out_hbm.at[pl.ds(base + c * _CHUNK, _CHUNK)])


def _sc_gather(ids, tab4, qmask):
    mesh = plsc.VectorSubcoreMesh(
        core_axis_name="c", subcore_axis_name="s",
        num_cores=_NC, num_subcores=_NS)
    f = pl.kernel(
        lambda *a: _sc_gather_body(qmask, *a),
        out_type=jax.ShapeDtypeStruct((BATCH, _GW), jnp.float32),
        mesh=mesh,
        scratch_types=(
            pltpu.VMEM((_B_PER_W,), jnp.int32),
            pltpu.VMEM((_CHUNK, _GW), jnp.float32),
            pltpu.VMEM((_CHUNK, _GW), jnp.float32),
            pltpu.SemaphoreType.DMA,
            pltpu.SemaphoreType.DMA,
        ),
        compiler_params=pltpu.CompilerParams(use_tc_tiling_on_sc=True),
    )
    return f(ids, tab4)


_MLP_BLK = 2048


def _unpack(packed, ids, ebits):
    u32 = jnp.uint32
    w = lax.bitcast_convert_type(packed, u32)
    half = lax.bitwise_and(lax.shift_right_logical(ids, ebits), 1)
    hi = jnp.bitwise_and(w, jnp.uint32(0xFFFF0000))
    lo = jnp.left_shift(w, jnp.uint32(16))
    return lax.bitcast_convert_type(jnp.where(half == 1, hi, lo), jnp.float32)


def _mlp_body(u4_ref, i4_ref, uid_ref, iid_ref,
              w1a_ref, w1b_ref, b1_ref, w2_ref, b2_ref,
              w3_ref, b3_ref, w4_ref, b4_ref, out_ref):
    f32 = jnp.float32
    uid = uid_ref[...]
    iid = iid_ref[...]
    su = lax.shift_right_logical(uid, _UE_LOG + 1)
    si = lax.shift_right_logical(iid, _IE_LOG + 1)
    u4 = _unpack(u4_ref[...].reshape(_MLP_BLK, _GW), uid, _UE_LOG)
    i4 = _unpack(i4_ref[...].reshape(_MLP_BLK, _GW), iid, _IE_LOG)
    lane_grp = lax.shift_right_logical(
        lax.broadcasted_iota(jnp.int32, (_MLP_BLK, _GW), 1), 5)
    um = jnp.where(lane_grp == su, u4, 0.0)
    im = jnp.where(lane_grp == si, i4, 0.0)
    h = (jnp.dot(um, w1a_ref[...], preferred_element_type=f32)
         + jnp.dot(im, w1b_ref[...], preferred_element_type=f32)
         + b1_ref[...])
    h = jnp.maximum(h, 0.0)
    h = jnp.dot(h, w2_ref[...], preferred_element_type=f32) + b2_ref[...]
    h = jnp.maximum(h, 0.0)
    h = jnp.dot(h, w3_ref[...], preferred_element_type=f32) + b3_ref[...]
    h = jnp.maximum(h, 0.0)
    z = jnp.dot(h, w4_ref[...], preferred_element_type=f32) + b4_ref[...]
    out_ref[...] = jnp.squeeze(jax.nn.sigmoid(z), axis=-1)


def _mlp(u4, i4, user_ids, item_ids, W1, b1, W2, b2, W3, b3, W4, b4):
    w1a = jnp.concatenate([W1[:EMBED_DIM]] * 4, axis=0)
    w1b = jnp.concatenate([W1[EMBED_DIM:]] * 4, axis=0)
    grid = BATCH // _MLP_BLK
    full = lambda a: pl.BlockSpec(a.shape, lambda b: (0,) * a.ndim)
    out = pl.pallas_call(
        _mlp_body,
        grid=(grid,),
        in_specs=[
            pl.BlockSpec((_MLP_BLK // 8, 8, _GW), lambda b: (b, 0, 0)),
            pl.BlockSpec((_MLP_BLK // 8, 8, _GW), lambda b: (b, 0, 0)),
            pl.BlockSpec((_MLP_BLK, 1), lambda b: (b, 0)),
            pl.BlockSpec((_MLP_BLK, 1), lambda b: (b, 0)),
            full(w1a), full(w1b),
            pl.BlockSpec((1, 64), lambda b: (0, 0)),
            full(W2),
            pl.BlockSpec((1, 32), lambda b: (0, 0)),
            full(W3),
            pl.BlockSpec((1, 16), lambda b: (0, 0)),
            full(W4),
            pl.BlockSpec((1, 1), lambda b: (0, 0)),
        ],
        out_specs=pl.BlockSpec((_MLP_BLK,), lambda b: (b,)),
        out_shape=jax.ShapeDtypeStruct((BATCH,), jnp.float32),
    )(u4.reshape(BATCH // 8, 8, _GW), i4.reshape(BATCH // 8, 8, _GW),
      user_ids.reshape(BATCH, 1), item_ids.reshape(BATCH, 1),
      w1a, w1b, b1.reshape(1, 64), W2, b2.reshape(1, 32),
      W3, b3.reshape(1, 16), W4, b4.reshape(1, 1))
    return out


def kernel(user_ids, item_ids, user_table, item_table,
           W1, b1, W2, b2, W3, b3, W4, b4):
    itab4 = _prepass(item_table.T, _IE_LOG, blk=2048)
    i4 = _sc_gather(item_ids, itab4, _IE - 1)
    utab4 = _prepass(user_table.T, _UE_LOG)
    u4 = _sc_gather(user_ids, utab4, _UE - 1)
    return _mlp(u4, i4, user_ids, item_ids, W1, b1, W2, b2, W3, b3, W4, b4)
